# baseline (speedup 1.0000x reference)
import functools

import jax
import jax.numpy as jnp
from jax import lax
from jax.experimental import pallas as pl
from jax.experimental.pallas import tpu as pltpu
from jax.experimental.pallas import tpu_sc as plsc

N = 10000
D = 128
H = 16
C = 40
E = 320000

NC = 2
NS = 16
NW = NC * NS
CH = 128
ROWS = 80
EP = NW * ROWS * CH
CHA = 1024
RA = EP // (NW * CHA)
NP = 10240
NPT = NP // NS
SROWS = 624
CSPLIT = (16, 16, 8)

_mesh = plsc.VectorSubcoreMesh(core_axis_name="c", subcore_axis_name="s")


def _sc_deg(dst_p, zblk):
  @functools.partial(
      pl.kernel,
      out_type=jax.ShapeDtypeStruct((NC, NP), jnp.float32),
      mesh=_mesh,
      scratch_types=[
          pltpu.VMEM((ROWS, CH), jnp.int32),
          pltpu.VMEM((CH,), jnp.float32),
          pltpu.VMEM_SHARED((NP,), jnp.float32),
          pltpu.SemaphoreType.DMA,
      ],
  )
  def k(dst_hbm, z_hbm, out_hbm, dstv, onesv, acc, sem):
    cid = lax.axis_index("c")
    sid = lax.axis_index("s")
    wid = sid * NC + cid
    pltpu.sync_copy(z_hbm.at[pl.ds(sid * NPT, NPT)], acc.at[pl.ds(sid * NPT, NPT)])
    for i in range(CH // 16):
      onesv[pl.ds(i * 16, 16)] = jnp.ones((16,), jnp.float32)
    pltpu.sync_copy(dst_hbm.at[pl.ds(wid * ROWS, ROWS)], dstv)
    plsc.subcore_barrier()

    def body(j, carry):
      pltpu.sync_copy(onesv, acc.at[dstv.at[j]], add=True)
      return carry

    lax.fori_loop(0, ROWS, body, 0)
    plsc.subcore_barrier()
    pltpu.sync_copy(acc.at[pl.ds(sid * NPT, NPT)],
                    out_hbm.at[cid, pl.ds(sid * NPT, NPT)])

  return k(dst_p, zblk)


def _sc_agg(F, stage_spmem):
  scratch = [
      pltpu.VMEM((RA, CHA), jnp.int32),
      pltpu.VMEM((RA, CHA), jnp.int32),
      pltpu.VMEM((CHA, F), jnp.float32),
      pltpu.VMEM((CHA, F), jnp.float32),
      pltpu.VMEM_SHARED((NP, F), jnp.float32),
      pltpu.VMEM_SHARED((N if stage_spmem else NS, F), jnp.float32),
      pltpu.SemaphoreType.DMA,
      pltpu.SemaphoreType.DMA,
      pltpu.SemaphoreType.DMA,
      pltpu.SemaphoreType.DMA,
  ]

  @functools.partial(
      pl.kernel,
      out_type=jax.ShapeDtypeStruct((NC, NP, F), jnp.float32),
      mesh=_mesh,
      compiler_params=pltpu.CompilerParams(use_tc_tiling_on_sc=False),
      scratch_types=scratch,
  )
  def k(src_hbm, dst_hbm, g_hbm, z_hbm, out_hbm, srcv, dstv, rows0, rows1,
        acc, gsh, gs0, gs1, ss0, ss1):
    cid = lax.axis_index("c")
    sid = lax.axis_index("s")
    wid = sid * NC + cid
    pltpu.sync_copy(z_hbm.at[pl.ds(sid * NPT, NPT)], acc.at[pl.ds(sid * NPT, NPT)])
    if stage_spmem:
      pltpu.sync_copy(g_hbm.at[pl.ds(sid * SROWS, SROWS)],
                      gsh.at[pl.ds(sid * SROWS, SROWS)])

      @pl.when(sid == 0)
      def _():
        pltpu.sync_copy(g_hbm.at[pl.ds(NS * SROWS, N - NS * SROWS)],
                        gsh.at[pl.ds(NS * SROWS, N - NS * SROWS)])

      gsrc = gsh
    else:
      gsrc = g_hbm
    pltpu.sync_copy(src_hbm.at[pl.ds(wid * RA, RA)], srcv)
    pltpu.sync_copy(dst_hbm.at[pl.ds(wid * RA, RA)], dstv)
    plsc.subcore_barrier()

    def gather(j, buf, sem):
      pltpu.async_copy(gsrc.at[srcv.at[j]], buf, sem)

    def scatter(j, buf, sem):
      pltpu.async_copy(buf, acc.at[dstv.at[j]], sem, add=True)

    def gwait(buf, sem):
      pltpu.make_async_copy(gsrc.at[srcv.at[0]], buf, sem).wait()

    def swait(buf, sem):
      pltpu.make_async_copy(buf, acc.at[dstv.at[0]], sem).wait()

    gather(0, rows0, gs0)
    gwait(rows0, gs0)
    scatter(0, rows0, ss0)
    gather(1, rows1, gs1)

    def body(j2, carry):
      j = 2 * j2 + 1
      gwait(rows1, gs1)
      scatter(j, rows1, ss1)
      swait(rows0, ss0)
      gather(j + 1, rows0, gs0)
      gwait(rows0, gs0)
      scatter(j + 1, rows0, ss0)
      swait(rows1, ss1)
      gather(j + 2, rows1, gs1)
      return carry

    lax.fori_loop(0, (RA - 2) // 2, body, 0)
    gwait(rows1, gs1)
    scatter(RA - 1, rows1, ss1)
    swait(rows0, ss0)
    swait(rows1, ss1)
    plsc.subcore_barrier()
    pltpu.sync_copy(acc.at[pl.ds(sid * NPT, NPT)],
                    out_hbm.at[cid, pl.ds(sid * NPT, NPT)])

  return k


def _sc_agg3():
  F = 16
  scratch = [
      pltpu.VMEM((RA, CHA), jnp.int32),
      pltpu.VMEM((RA, CHA), jnp.int32),
      pltpu.VMEM((CHA, F), jnp.float32),
      pltpu.VMEM((CHA, F), jnp.float32),
      pltpu.VMEM_SHARED((NP, F), jnp.float32),
      pltpu.VMEM_SHARED((N, F), jnp.float32),
      pltpu.SemaphoreType.DMA,
      pltpu.SemaphoreType.DMA,
      pltpu.SemaphoreType.DMA,
      pltpu.SemaphoreType.DMA,
  ]

  @functools.partial(
      pl.kernel,
      out_type=[jax.ShapeDtypeStruct((NC, NP, F), jnp.float32)] * 3,
      mesh=_mesh,
      compiler_params=pltpu.CompilerParams(use_tc_tiling_on_sc=False),
      scratch_types=scratch,
  )
  def k(src_hbm, dst_hbm, ga_hbm, gb_hbm, gc_hbm, z_hbm,
        outa_hbm, outb_hbm, outc_hbm, srcv, dstv, rows0, rows1,
        acc, gsh, gs0, gs1, ss0, ss1):
    cid = lax.axis_index("c")
    sid = lax.axis_index("s")
    wid = sid * NC + cid
    g_phases = (ga_hbm, gb_hbm, gc_hbm)
    out_phases = (outa_hbm, outb_hbm, outc_hbm)
    pltpu.sync_copy(src_hbm.at[pl.ds(wid * RA, RA)], srcv)
    pltpu.sync_copy(dst_hbm.at[pl.ds(wid * RA, RA)], dstv)

    def gather(j, buf, sem):
      pltpu.async_copy(gsh.at[srcv.at[j]], buf, sem)

    def scatter(j, buf, sem):
      pltpu.async_copy(buf, acc.at[dstv.at[j]], sem, add=True)

    def gwait(buf, sem):
      pltpu.make_async_copy(gsh.at[srcv.at[0]], buf, sem).wait()

    def swait(buf, sem):
      pltpu.make_async_copy(buf, acc.at[dstv.at[0]], sem).wait()

    for p in range(3):
      pltpu.sync_copy(z_hbm.at[pl.ds(sid * NPT, NPT)],
                      acc.at[pl.ds(sid * NPT, NPT)])
      pltpu.sync_copy(g_phases[p].at[pl.ds(sid * SROWS, SROWS)],
                      gsh.at[pl.ds(sid * SROWS, SROWS)])

      @pl.when(sid == 0)
      def _():
        pltpu.sync_copy(g_phases[p].at[pl.ds(NS * SROWS, N - NS * SROWS)],
                        gsh.at[pl.ds(NS * SROWS, N - NS * SROWS)])

      plsc.subcore_barrier()

      gather(0, rows0, gs0)
      gwait(rows0, gs0)
      scatter(0, rows0, ss0)
      gather(1, rows1, gs1)

      def body(j2, carry):
        j = 2 * j2 + 1
        gwait(rows1, gs1)
        scatter(j, rows1, ss1)
        swait(rows0, ss0)
        gather(j + 1, rows0, gs0)
        gwait(rows0, gs0)
        scatter(j + 1, rows0, ss0)
        swait(rows1, ss1)
        gather(j + 2, rows1, gs1)
        return carry

      lax.fori_loop(0, (RA - 2) // 2, body, 0)
      gwait(rows1, gs1)
      scatter(RA - 1, rows1, ss1)
      swait(rows0, ss0)
      swait(rows1, ss1)
      plsc.subcore_barrier()
      pltpu.sync_copy(acc.at[pl.ds(sid * NPT, NPT)],
                      out_phases[p].at[cid, pl.ds(sid * NPT, NPT)])
      if p < 2:
        plsc.subcore_barrier()

  return k


def _dinv_col(degp_ref):
  deg = degp_ref[0, :N] + degp_ref[1, :N] + 1.0
  return lax.rsqrt(deg)[:, None]


def _tc_a(degp, x, W1):
  def body(degp_ref, x_ref, w_ref, g1_ref, dinv_ref):
    dinv = _dinv_col(degp_ref)
    h = jnp.dot(x_ref[...], w_ref[...], preferred_element_type=jnp.float32)
    g1_ref[...] = h * dinv
    dinv_ref[...] = dinv

  return pl.pallas_call(
      body, out_shape=[jax.ShapeDtypeStruct((N, H), jnp.float32),
                       jax.ShapeDtypeStruct((N, 1), jnp.float32)])(
          degp, x, W1)


def _tc_b(dinvc, acc1p, g1, W2, b1):
  def body(dinv_ref, accp_ref, g1_ref, w_ref, b_ref, *g2_refs):
    dinv = dinv_ref[...]
    agg = accp_ref[0, :N, :] + accp_ref[1, :N, :] + g1_ref[...]
    out1 = jnp.maximum(agg * dinv + b_ref[...][None, :], 0.0)
    g2 = jnp.dot(out1, w_ref[...], preferred_element_type=jnp.float32) * dinv
    g2_refs[0][...] = g2[:, :16]
    g2_refs[1][...] = g2[:, 16:32]
    g2_refs[2][...] = jnp.concatenate(
        [g2[:, 32:40], jnp.zeros((N, 8), jnp.float32)], axis=1)

  return pl.pallas_call(
      body, out_shape=[jax.ShapeDtypeStruct((N, 16), jnp.float32)] * 3)(
          dinvc, acc1p, g1, W2, b1)


def _tc_c(dinvc, acc2ps, g2s, b2):
  BN = 2000

  def body(dinv_ref, *refs):
    accp_refs = refs[:3]
    g2_refs = refs[3:6]
    b_ref = refs[6]
    out_ref = refs[7]
    dinv = dinv_ref[...]
    aggs = [ap[0] + ap[1] + gp[...] for ap, gp in zip(accp_refs, g2_refs)]
    out2 = jnp.concatenate(aggs, axis=1)[:, :C] * dinv + b_ref[...][None, :]
    z = out2 - jnp.max(out2, axis=1, keepdims=True)
    out_ref[...] = z - jnp.log(jnp.sum(jnp.exp(z), axis=1, keepdims=True))

  in_specs = [pl.BlockSpec((BN, 1), lambda i: (i, 0))]
  in_specs += [pl.BlockSpec((2, BN, 16), lambda i: (0, i, 0))] * 3
  in_specs += [pl.BlockSpec((BN, 16), lambda i: (i, 0))] * 3
  in_specs += [pl.BlockSpec((C,), lambda i: (0,))]
  return pl.pallas_call(
      body,
      grid=(N // BN,),
      in_specs=in_specs,
      out_specs=pl.BlockSpec((BN, C), lambda i: (i, 0)),
      out_shape=jax.ShapeDtypeStruct((N, C), jnp.float32))(
          dinvc, *acc2ps, *g2s, b2)


def kernel(x, edge_index, W1, b1, W2, b2):
  pad = EP - E
  src_flat = jnp.concatenate([edge_index[0], jnp.zeros((pad,), jnp.int32)])
  dst_flat = jnp.concatenate([edge_index[1], jnp.full((pad,), NP - 1, jnp.int32)])
  dst_p = dst_flat.reshape(EP // CH, CH)
  src_a = src_flat.reshape(EP // CHA, CHA)
  dst_a = dst_flat.reshape(EP // CHA, CHA)

  z1 = jnp.zeros((NP,), jnp.float32)
  zH = jnp.zeros((NP, H), jnp.float32)

  degp = _sc_deg(dst_p, z1)
  g1, dinvc = _tc_a(degp, x, W1)
  acc1p = _sc_agg(H, True)(src_a, dst_a, g1, zH)
  g2s = _tc_b(dinvc, acc1p, g1, W2, b1)
  acc2ps = _sc_agg3()(src_a, dst_a, g2s[0], g2s[1], g2s[2], zH)
  return _tc_c(dinvc, acc2ps, g2s, b2)


# speedup vs baseline: 1.0282x; 1.0282x over previous
import functools

import jax
import jax.numpy as jnp
from jax import lax
from jax.experimental import pallas as pl
from jax.experimental.pallas import tpu as pltpu
from jax.experimental.pallas import tpu_sc as plsc

N = 10000
D = 128
H = 16
C = 40
E = 320000

NC = 2
NS = 16
NW = NC * NS
CH = 128
ROWS = 80
EP = NW * ROWS * CH
CHA = 1024
RA = EP // (NW * CHA)
NP = 10240
NPT = NP // NS
SROWS = 624
CSPLIT = (16, 16, 8)

_mesh = plsc.VectorSubcoreMesh(core_axis_name="c", subcore_axis_name="s")


def _sc_deg(dst_p, zblk):
  @functools.partial(
      pl.kernel,
      out_type=jax.ShapeDtypeStruct((NC, NP), jnp.float32),
      mesh=_mesh,
      scratch_types=[
          pltpu.VMEM((ROWS, CH), jnp.int32),
          pltpu.VMEM((CH,), jnp.float32),
          pltpu.VMEM_SHARED((NP,), jnp.float32),
          pltpu.SemaphoreType.DMA,
      ],
  )
  def k(dst_hbm, z_hbm, out_hbm, dstv, onesv, acc, sem):
    cid = lax.axis_index("c")
    sid = lax.axis_index("s")
    wid = sid * NC + cid
    pltpu.sync_copy(z_hbm.at[pl.ds(sid * NPT, NPT)], acc.at[pl.ds(sid * NPT, NPT)])
    for i in range(CH // 16):
      onesv[pl.ds(i * 16, 16)] = jnp.ones((16,), jnp.float32)
    pltpu.sync_copy(dst_hbm.at[pl.ds(wid * ROWS, ROWS)], dstv)
    plsc.subcore_barrier()

    def body(j, carry):
      pltpu.sync_copy(onesv, acc.at[dstv.at[j]], add=True)
      return carry

    lax.fori_loop(0, ROWS, body, 0)
    plsc.subcore_barrier()
    pltpu.sync_copy(acc.at[pl.ds(sid * NPT, NPT)],
                    out_hbm.at[cid, pl.ds(sid * NPT, NPT)])

  return k(dst_p, zblk)


def _sc_agg(F, stage_spmem):
  scratch = [
      pltpu.VMEM((RA, CHA), jnp.int32),
      pltpu.VMEM((RA, CHA), jnp.int32),
      pltpu.VMEM((CHA, F), jnp.float32),
      pltpu.VMEM((CHA, F), jnp.float32),
      pltpu.VMEM_SHARED((NP, F), jnp.float32),
      pltpu.VMEM_SHARED((N if stage_spmem else NS, F), jnp.float32),
      pltpu.SemaphoreType.DMA,
      pltpu.SemaphoreType.DMA,
      pltpu.SemaphoreType.DMA,
      pltpu.SemaphoreType.DMA,
  ]

  @functools.partial(
      pl.kernel,
      out_type=jax.ShapeDtypeStruct((NC, NP, F), jnp.float32),
      mesh=_mesh,
      compiler_params=pltpu.CompilerParams(use_tc_tiling_on_sc=False),
      scratch_types=scratch,
  )
  def k(src_hbm, dst_hbm, g_hbm, z_hbm, out_hbm, srcv, dstv, rows0, rows1,
        acc, gsh, gs0, gs1, ss0, ss1):
    cid = lax.axis_index("c")
    sid = lax.axis_index("s")
    wid = sid * NC + cid
    pltpu.sync_copy(z_hbm.at[pl.ds(sid * NPT, NPT)], acc.at[pl.ds(sid * NPT, NPT)])
    if stage_spmem:
      pltpu.sync_copy(g_hbm.at[pl.ds(sid * SROWS, SROWS)],
                      gsh.at[pl.ds(sid * SROWS, SROWS)])

      @pl.when(sid == 0)
      def _():
        pltpu.sync_copy(g_hbm.at[pl.ds(NS * SROWS, N - NS * SROWS)],
                        gsh.at[pl.ds(NS * SROWS, N - NS * SROWS)])

      gsrc = gsh
    else:
      gsrc = g_hbm
    pltpu.sync_copy(src_hbm.at[pl.ds(wid * RA, RA)], srcv)
    pltpu.sync_copy(dst_hbm.at[pl.ds(wid * RA, RA)], dstv)
    plsc.subcore_barrier()

    def gather(j, buf, sem):
      pltpu.async_copy(gsrc.at[srcv.at[j]], buf, sem)

    def scatter(j, buf, sem):
      pltpu.async_copy(buf, acc.at[dstv.at[j]], sem, add=True)

    def gwait(buf, sem):
      pltpu.make_async_copy(gsrc.at[srcv.at[0]], buf, sem).wait()

    def swait(buf, sem):
      pltpu.make_async_copy(buf, acc.at[dstv.at[0]], sem).wait()

    gather(0, rows0, gs0)
    gwait(rows0, gs0)
    scatter(0, rows0, ss0)
    gather(1, rows1, gs1)

    def body(j2, carry):
      j = 2 * j2 + 1
      gwait(rows1, gs1)
      scatter(j, rows1, ss1)
      swait(rows0, ss0)
      gather(j + 1, rows0, gs0)
      gwait(rows0, gs0)
      scatter(j + 1, rows0, ss0)
      swait(rows1, ss1)
      gather(j + 2, rows1, gs1)
      return carry

    lax.fori_loop(0, (RA - 2) // 2, body, 0)
    gwait(rows1, gs1)
    scatter(RA - 1, rows1, ss1)
    swait(rows0, ss0)
    swait(rows1, ss1)
    plsc.subcore_barrier()
    pltpu.sync_copy(acc.at[pl.ds(sid * NPT, NPT)],
                    out_hbm.at[cid, pl.ds(sid * NPT, NPT)])

  return k


def _dinv_col(degp_ref):
  deg = degp_ref[0, :N] + degp_ref[1, :N] + 1.0
  return lax.rsqrt(deg)[:, None]


def _tc_a(degp, x, W1):
  def body(degp_ref, x_ref, w_ref, g1_ref, dinv_ref):
    dinv = _dinv_col(degp_ref)
    h = jnp.dot(x_ref[...], w_ref[...], preferred_element_type=jnp.float32)
    g1_ref[...] = h * dinv
    dinv_ref[...] = dinv

  return pl.pallas_call(
      body, out_shape=[jax.ShapeDtypeStruct((N, H), jnp.float32),
                       jax.ShapeDtypeStruct((N, 1), jnp.float32)])(
          degp, x, W1)


def _tc_b(dinvc, acc1p, g1, W2, b1):
  def body(dinv_ref, accp_ref, g1_ref, w_ref, b_ref, *g2_refs):
    dinv = dinv_ref[...]
    agg = accp_ref[0, :N, :] + accp_ref[1, :N, :] + g1_ref[...]
    out1 = jnp.maximum(agg * dinv + b_ref[...][None, :], 0.0)
    g2 = jnp.dot(out1, w_ref[...], preferred_element_type=jnp.float32) * dinv
    lo = 0
    for f, ref in zip(CSPLIT, g2_refs):
      ref[...] = g2[:, lo:lo + f]
      lo += f

  return pl.pallas_call(
      body, out_shape=[jax.ShapeDtypeStruct((N, f), jnp.float32)
                       for f in CSPLIT])(dinvc, acc1p, g1, W2, b1)


def _tc_c(dinvc, acc2ps, g2s, b2):
  BN = 2000

  def body(dinv_ref, *refs):
    accp_refs = refs[:len(CSPLIT)]
    g2_refs = refs[len(CSPLIT):2 * len(CSPLIT)]
    b_ref = refs[2 * len(CSPLIT)]
    out_ref = refs[2 * len(CSPLIT) + 1]
    dinv = dinv_ref[...]
    aggs = [ap[0] + ap[1] + gp[...] for ap, gp in zip(accp_refs, g2_refs)]
    out2 = jnp.concatenate(aggs, axis=1) * dinv + b_ref[...][None, :]
    z = out2 - jnp.max(out2, axis=1, keepdims=True)
    out_ref[...] = z - jnp.log(jnp.sum(jnp.exp(z), axis=1, keepdims=True))

  in_specs = [pl.BlockSpec((BN, 1), lambda i: (i, 0))]
  in_specs += [pl.BlockSpec((2, BN, f), lambda i: (0, i, 0)) for f in CSPLIT]
  in_specs += [pl.BlockSpec((BN, f), lambda i: (i, 0)) for f in CSPLIT]
  in_specs += [pl.BlockSpec((C,), lambda i: (0,))]
  return pl.pallas_call(
      body,
      grid=(N // BN,),
      in_specs=in_specs,
      out_specs=pl.BlockSpec((BN, C), lambda i: (i, 0)),
      out_shape=jax.ShapeDtypeStruct((N, C), jnp.float32))(
          dinvc, *acc2ps, *g2s, b2)


def kernel(x, edge_index, W1, b1, W2, b2):
  pad = EP - E
  src_flat = jnp.concatenate([edge_index[0], jnp.zeros((pad,), jnp.int32)])
  dst_flat = jnp.concatenate([edge_index[1], jnp.full((pad,), NP - 1, jnp.int32)])
  dst_p = dst_flat.reshape(EP // CH, CH)
  src_a = src_flat.reshape(EP // CHA, CHA)
  dst_a = dst_flat.reshape(EP // CHA, CHA)

  z1 = jnp.zeros((NP,), jnp.float32)
  zH = jnp.zeros((NP, H), jnp.float32)

  degp = _sc_deg(dst_p, z1)
  g1, dinvc = _tc_a(degp, x, W1)
  acc1p = _sc_agg(H, True)(src_a, dst_a, g1, zH)
  g2s = _tc_b(dinvc, acc1p, g1, W2, b1)
  acc2ps = [
      _sc_agg(f, True)(src_a, dst_a, g2f, jnp.zeros((NP, f), jnp.float32))
      for f, g2f in zip(CSPLIT, g2s)
  ]
  return _tc_c(dinvc, acc2ps, g2s, b2)
